# baseline (speedup 1.0000x reference)
import functools

import jax
import jax.numpy as jnp
from jax.experimental import pallas as pl
from jax.experimental.pallas import tpu as pltpu


def _probe(H, kc, vc, kv, vv, ko, vo, sem):
    def start(h, _):
        pltpu.make_async_copy(kc.at[0, h], ko.at[0, h], sem).start()
        pltpu.make_async_copy(vc.at[0, h], vo.at[0, h], sem).start()
        return 0

    jax.lax.fori_loop(0, H, start, 0)

    def wait(h, _):
        pltpu.make_async_copy(kc.at[0, h], ko.at[0, h], sem).wait()
        pltpu.make_async_copy(vc.at[0, h], vo.at[0, h], sem).wait()
        return 0

    jax.lax.fori_loop(0, H, wait, 0)


@jax.jit
def kernel(k_cache, v_cache, k_val, v_val, input_pos):
    B, H, BUF, D = k_cache.shape
    any_spec = pl.BlockSpec(memory_space=pl.ANY)
    k_new, v_new = pl.pallas_call(
        functools.partial(_probe, H),
        in_specs=[any_spec] * 4,
        out_specs=[any_spec] * 2,
        out_shape=[
            jax.ShapeDtypeStruct(k_cache.shape, k_cache.dtype),
            jax.ShapeDtypeStruct(v_cache.shape, v_cache.dtype),
        ],
        scratch_shapes=[pltpu.SemaphoreType.DMA],
    )(k_cache, v_cache, k_val, v_val)
    return (k_new, v_new)


# speedup vs baseline: 40.4379x; 40.4379x over previous
import functools

import jax
import jax.numpy as jnp
from jax import lax
from jax.experimental import pallas as pl
from jax.experimental.pallas import tpu as pltpu
from jax.experimental.pallas import tpu_sc as plsc

_NC = 2
_NS = 16


def _copy_kernel(kc_ref, vc_ref, ko_ref, vo_ref):
    ko_ref[...] = kc_ref[...]
    vo_ref[...] = vc_ref[...]


def _sc_scatter_body(S, BUF, D, C,
                     ko_ref, vo_ref, kv_ref, vv_ref, p_ref,
                     rows_ref, idx_ref, pvmem_ref, sem):
    wid = lax.axis_index("s") * _NC + lax.axis_index("c")

    pltpu.sync_copy(p_ref, pvmem_ref)
    p = pvmem_ref[...][0]

    def build_idx(i, base):
        vec = base + lax.iota(jnp.int32, 16) + i * 16
        vec = jnp.where(vec >= BUF, vec - BUF, vec)
        idx_ref[pl.ds(i * 16, 16)] = vec
        return base

    for c in range(S // C):
        lax.fori_loop(0, C // 16, build_idx, p + c * C)
        pltpu.sync_copy(kv_ref.at[0, wid, pl.ds(c * C, C)], rows_ref)
        pltpu.async_copy(rows_ref, ko_ref.at[0, wid].at[idx_ref], sem).wait()
        pltpu.sync_copy(vv_ref.at[0, wid, pl.ds(c * C, C)], rows_ref)
        pltpu.async_copy(rows_ref, vo_ref.at[0, wid].at[idx_ref], sem).wait()


@jax.jit
def kernel(k_cache, v_cache, k_val, v_val, input_pos):
    B, H, BUF, D = k_cache.shape
    S = k_val.shape[2]
    Rb = 8192
    C = 512

    p = jnp.asarray(input_pos, jnp.int32).reshape((1,)) % BUF

    spec = pl.BlockSpec((1, 1, Rb, D), lambda h, j: (0, h, j, 0))
    k0, v0 = pl.pallas_call(
        _copy_kernel,
        grid=(H, BUF // Rb),
        in_specs=[spec, spec],
        out_specs=[spec, spec],
        out_shape=[
            jax.ShapeDtypeStruct(k_cache.shape, k_cache.dtype),
            jax.ShapeDtypeStruct(v_cache.shape, v_cache.dtype),
        ],
        compiler_params=pltpu.CompilerParams(
            dimension_semantics=("arbitrary", "arbitrary"),
        ),
    )(k_cache, v_cache)

    rk = jax.new_ref(k0)
    rv = jax.new_ref(v0)

    mesh = plsc.VectorSubcoreMesh(core_axis_name="c", subcore_axis_name="s")
    scatter = pl.kernel(
        functools.partial(_sc_scatter_body, S, BUF, D, C),
        mesh=mesh,
        scratch_types=[
            pltpu.VMEM((C, D), jnp.float32),
            pltpu.VMEM((C,), jnp.int32),
            pltpu.VMEM((16,), jnp.int32),
            pltpu.SemaphoreType.DMA,
        ],
    )
    scatter(rk, rv, k_val, v_val, jnp.broadcast_to(p, (16,)))

    return (jax.freeze(rk), jax.freeze(rv))
